# Initial kernel scaffold; baseline (speedup 1.0000x reference)
import jax
import jax.numpy as jnp
from jax.experimental import pallas as pl


def kernel(x, emb_table, W_epoch, W_cfg, b_cfg):
    raise NotImplementedError("write your pallas kernel here")


# speedup vs baseline: 1.7204x; 1.7204x over previous
import functools
import math

import jax
import jax.numpy as jnp
from jax import lax
from jax.experimental import pallas as pl
from jax.experimental.pallas import tpu as pltpu
from jax.experimental.pallas import tpu_sc as plsc

IN_DIM = 34
OUT_DIM = 256
SEQ = 2048
BATCH = 128
N_EMB = 1001
SB = SEQ * BATCH


def _make_sc_gather(sb, d):
    info = plsc.get_sparse_core_info()
    nc, ns = info.num_cores, info.num_subcores
    nw = nc * ns
    bpw = sb // nw
    ch = 128
    nch = bpw // ch
    mesh = plsc.VectorSubcoreMesh(core_axis_name="c", subcore_axis_name="s")

    @functools.partial(
        pl.kernel,
        mesh=mesh,
        out_type=jax.ShapeDtypeStruct((sb, d), jnp.float32),
        scratch_types=[
            pltpu.VMEM((ch,), jnp.int32),
            pltpu.VMEM((ch, d), jnp.float32),
            pltpu.SemaphoreType.DMA,
        ],
    )
    def gather_k(idx_hbm, table_hbm, out_hbm, idx_v, rows_v, sem):
        wid = lax.axis_index("s") * nc + lax.axis_index("c")
        base = wid * bpw

        def body(i, carry):
            off = base + i * ch
            pltpu.sync_copy(idx_hbm.at[pl.ds(off, ch)], idx_v)
            pltpu.async_copy(table_hbm.at[idx_v], rows_v, sem).wait()
            pltpu.sync_copy(rows_v, out_hbm.at[pl.ds(off, ch)])
            return carry

        lax.fori_loop(0, nch, body, 0)

    return gather_k


def _tc_body(x_ref, g_ref, w_ref, b_ref, o_ref):
    o_ref[...] = (
        g_ref[...]
        + jnp.dot(x_ref[...], w_ref[...], preferred_element_type=jnp.float32)
        + b_ref[...]
    )


def kernel(x, emb_table, W_epoch, W_cfg, b_cfg):
    x_flat = x.reshape(SB, IN_DIM)
    ids = x_flat[:, 0].astype(jnp.int32)

    inv_std = 1.0 / math.sqrt(1.0 / 12.0)
    w_epoch_row = (W_epoch[:, 0] * inv_std)[None, :]
    b_all = (b_cfg - 0.5 * inv_std * W_epoch[:, 0])[None, :]
    w34 = jnp.concatenate(
        [jnp.zeros((1, OUT_DIM), jnp.float32), w_epoch_row, W_cfg.T], axis=0
    )

    gathered = _make_sc_gather(SB, OUT_DIM)(ids, emb_table)

    bt = 1024
    out = pl.pallas_call(
        _tc_body,
        grid=(SB // bt,),
        in_specs=[
            pl.BlockSpec((bt, IN_DIM), lambda i: (i, 0)),
            pl.BlockSpec((bt, OUT_DIM), lambda i: (i, 0)),
            pl.BlockSpec((IN_DIM, OUT_DIM), lambda i: (0, 0)),
            pl.BlockSpec((1, OUT_DIM), lambda i: (0, 0)),
        ],
        out_specs=pl.BlockSpec((bt, OUT_DIM), lambda i: (i, 0)),
        out_shape=jax.ShapeDtypeStruct((SB, OUT_DIM), jnp.float32),
    )(x_flat, gathered, w34, b_all)

    return out.reshape(SEQ, BATCH, OUT_DIM)


# speedup vs baseline: 2.0458x; 1.1891x over previous
import functools
import math

import jax
import jax.numpy as jnp
from jax import lax
from jax.experimental import pallas as pl
from jax.experimental.pallas import tpu as pltpu
from jax.experimental.pallas import tpu_sc as plsc

IN_DIM = 34
OUT_DIM = 256
SEQ = 2048
BATCH = 128
N_EMB = 1001
SB = SEQ * BATCH


def _make_sc_gather(sb, d):
    info = plsc.get_sparse_core_info()
    nc, ns = info.num_cores, info.num_subcores
    nw = nc * ns
    bpw = sb // nw
    ch = 128
    nch = bpw // ch
    mesh = plsc.VectorSubcoreMesh(core_axis_name="c", subcore_axis_name="s")

    @functools.partial(
        pl.kernel,
        mesh=mesh,
        out_type=jax.ShapeDtypeStruct((sb, d), jnp.int32),
        scratch_types=[
            pltpu.VMEM((ch,), jnp.int32),
            pltpu.VMEM((ch, d), jnp.int32),
            pltpu.SemaphoreType.DMA,
        ],
    )
    def gather_k(idx_hbm, table_hbm, out_hbm, idx_v, rows_v, sem):
        wid = lax.axis_index("s") * nc + lax.axis_index("c")
        base = wid * bpw

        def body(i, carry):
            off = base + i * ch
            pltpu.sync_copy(idx_hbm.at[pl.ds(off, ch)], idx_v)
            pltpu.async_copy(table_hbm.at[idx_v], rows_v, sem).wait()
            pltpu.sync_copy(rows_v, out_hbm.at[pl.ds(off, ch)])
            return carry

        lax.fori_loop(0, nch, body, 0)

    return gather_k


def _tc_body(x_ref, g_ref, w_ref, b_ref, o_ref):
    dense = (
        jnp.dot(x_ref[...], w_ref[...], preferred_element_type=jnp.float32)
        + b_ref[...]
    )
    gi = g_ref[...]
    hi = jax.lax.bitcast_convert_type(
        jnp.bitwise_and(gi, jnp.int32(-65536)), jnp.float32
    )
    lo = jax.lax.bitcast_convert_type(jnp.left_shift(gi, 16), jnp.float32)
    o_ref[...] = dense + jnp.concatenate([hi, lo], axis=1)


def kernel(x, emb_table, W_epoch, W_cfg, b_cfg):
    x_flat = x.reshape(SB, IN_DIM)
    ids = x_flat[:, 0].astype(jnp.int32)
    t16 = emb_table.astype(jnp.bfloat16)
    au = jax.lax.bitcast_convert_type(t16[:, : OUT_DIM // 2], jnp.uint16)
    bu = jax.lax.bitcast_convert_type(t16[:, OUT_DIM // 2 :], jnp.uint16)
    table_pk = jax.lax.bitcast_convert_type(
        (au.astype(jnp.uint32) << 16) | bu.astype(jnp.uint32), jnp.int32
    )

    inv_std = 1.0 / math.sqrt(1.0 / 12.0)
    w_epoch_row = (W_epoch[:, 0] * inv_std)[None, :]
    b_all = (b_cfg - 0.5 * inv_std * W_epoch[:, 0])[None, :]
    w34 = jnp.concatenate(
        [jnp.zeros((1, OUT_DIM), jnp.float32), w_epoch_row, W_cfg.T], axis=0
    )

    gathered = _make_sc_gather(SB, OUT_DIM // 2)(ids, table_pk)

    bt = 1024
    out = pl.pallas_call(
        _tc_body,
        grid=(SB // bt,),
        in_specs=[
            pl.BlockSpec((bt, IN_DIM), lambda i: (i, 0)),
            pl.BlockSpec((bt, OUT_DIM // 2), lambda i: (i, 0)),
            pl.BlockSpec((IN_DIM, OUT_DIM), lambda i: (0, 0)),
            pl.BlockSpec((1, OUT_DIM), lambda i: (0, 0)),
        ],
        out_specs=pl.BlockSpec((bt, OUT_DIM), lambda i: (i, 0)),
        out_shape=jax.ShapeDtypeStruct((SB, OUT_DIM), jnp.float32),
    )(x_flat, gathered, w34, b_all)

    return out.reshape(SEQ, BATCH, OUT_DIM)


# speedup vs baseline: 2.1646x; 1.0581x over previous
import functools
import math

import jax
import jax.numpy as jnp
from jax import lax
from jax.experimental import pallas as pl
from jax.experimental.pallas import tpu as pltpu
from jax.experimental.pallas import tpu_sc as plsc

IN_DIM = 34
OUT_DIM = 256
SEQ = 2048
BATCH = 128
N_EMB = 1001
SB = SEQ * BATCH


def _make_sc_gather(sb, d):
    info = plsc.get_sparse_core_info()
    nc, ns = info.num_cores, info.num_subcores
    nw = nc * ns
    bpw = sb // nw
    ch = 128
    nch = bpw // ch
    mesh = plsc.VectorSubcoreMesh(core_axis_name="c", subcore_axis_name="s")

    nbuf = 4
    ngrp = nch // nbuf

    @functools.partial(
        pl.kernel,
        mesh=mesh,
        out_type=jax.ShapeDtypeStruct((sb, d), jnp.int32),
        scratch_types=[
            pltpu.VMEM((nbuf, ch), jnp.int32),
            pltpu.VMEM((nbuf, ch, d), jnp.int32),
            pltpu.SemaphoreType.DMA((nbuf,)),
            pltpu.SemaphoreType.DMA((nbuf,)),
            pltpu.SemaphoreType.DMA((nbuf,)),
        ],
    )
    def gather_k(idx_hbm, table_hbm, out_hbm, idx_v, rows_v, sem_i, sem_g, sem_w):
        wid = lax.axis_index("s") * nc + lax.axis_index("c")
        base = wid * bpw

        def idx_slice(c):
            return idx_hbm.at[pl.ds(base + c * ch, ch)]

        def out_slice(c):
            return out_hbm.at[pl.ds(base + c * ch, ch)]

        for b in range(nbuf - 1):
            pltpu.async_copy(idx_slice(b), idx_v.at[b], sem_i.at[b])

        def group(g, carry):
            for b in range(nbuf):
                i = g * nbuf + b
                bp = (b - 1) % nbuf
                @pl.when(i >= nbuf)
                def _():
                    pltpu.make_async_copy(
                        rows_v.at[b], out_slice(i - nbuf), sem_w.at[b]
                    ).wait()

                pltpu.make_async_copy(idx_slice(i), idx_v.at[b], sem_i.at[b]).wait()
                pltpu.async_copy(
                    table_hbm.at[idx_v.at[b]], rows_v.at[b], sem_g.at[b]
                )

                @pl.when(i >= 1)
                def _():
                    pltpu.make_async_copy(
                        table_hbm.at[idx_v.at[bp]], rows_v.at[bp], sem_g.at[bp]
                    ).wait()
                    pltpu.async_copy(rows_v.at[bp], out_slice(i - 1), sem_w.at[bp])

                @pl.when(i + nbuf - 1 < nch)
                def _():
                    pltpu.async_copy(
                        idx_slice(i + nbuf - 1), idx_v.at[bp], sem_i.at[bp]
                    )

            return carry

        lax.fori_loop(0, ngrp, group, 0)

        blast = (nch - 1) % nbuf
        pltpu.make_async_copy(
            table_hbm.at[idx_v.at[blast]], rows_v.at[blast], sem_g.at[blast]
        ).wait()
        pltpu.async_copy(rows_v.at[blast], out_slice(nch - 1), sem_w.at[blast])
        for b in range(nbuf):
            pltpu.make_async_copy(
                rows_v.at[b], out_slice(nch - nbuf + b), sem_w.at[b]
            ).wait()

    return gather_k


def _tc_body(x_ref, g_ref, w_ref, b_ref, o_ref):
    dense = (
        jnp.dot(x_ref[...], w_ref[...], preferred_element_type=jnp.float32)
        + b_ref[...]
    )
    gi = g_ref[...]
    hi = jax.lax.bitcast_convert_type(
        jnp.bitwise_and(gi, jnp.int32(-65536)), jnp.float32
    )
    lo = jax.lax.bitcast_convert_type(jnp.left_shift(gi, 16), jnp.float32)
    o_ref[...] = dense + jnp.concatenate([hi, lo], axis=1)


def kernel(x, emb_table, W_epoch, W_cfg, b_cfg):
    x_flat = x.reshape(SB, IN_DIM)
    ids = x_flat[:, 0].astype(jnp.int32)
    t16 = emb_table.astype(jnp.bfloat16)
    au = jax.lax.bitcast_convert_type(t16[:, : OUT_DIM // 2], jnp.uint16)
    bu = jax.lax.bitcast_convert_type(t16[:, OUT_DIM // 2 :], jnp.uint16)
    table_pk = jax.lax.bitcast_convert_type(
        (au.astype(jnp.uint32) << 16) | bu.astype(jnp.uint32), jnp.int32
    )

    inv_std = 1.0 / math.sqrt(1.0 / 12.0)
    w_epoch_row = (W_epoch[:, 0] * inv_std)[None, :]
    b_all = (b_cfg - 0.5 * inv_std * W_epoch[:, 0])[None, :]
    w34 = jnp.concatenate(
        [jnp.zeros((1, OUT_DIM), jnp.float32), w_epoch_row, W_cfg.T], axis=0
    )

    gathered = _make_sc_gather(SB, OUT_DIM // 2)(ids, table_pk)

    bt = 1024
    out = pl.pallas_call(
        _tc_body,
        grid=(SB // bt,),
        in_specs=[
            pl.BlockSpec((bt, IN_DIM), lambda i: (i, 0)),
            pl.BlockSpec((bt, OUT_DIM // 2), lambda i: (i, 0)),
            pl.BlockSpec((IN_DIM, OUT_DIM), lambda i: (0, 0)),
            pl.BlockSpec((1, OUT_DIM), lambda i: (0, 0)),
        ],
        out_specs=pl.BlockSpec((bt, OUT_DIM), lambda i: (i, 0)),
        out_shape=jax.ShapeDtypeStruct((SB, OUT_DIM), jnp.float32),
    )(x_flat, gathered, w34, b_all)

    return out.reshape(SEQ, BATCH, OUT_DIM)


# speedup vs baseline: 2.1657x; 1.0005x over previous
import functools
import math

import jax
import jax.numpy as jnp
from jax import lax
from jax.experimental import pallas as pl
from jax.experimental.pallas import tpu as pltpu
from jax.experimental.pallas import tpu_sc as plsc

IN_DIM = 34
OUT_DIM = 256
SEQ = 2048
BATCH = 128
N_EMB = 1001
SB = SEQ * BATCH


def _make_sc_gather(sb, d):
    info = plsc.get_sparse_core_info()
    nc, ns = info.num_cores, info.num_subcores
    nw = nc * ns
    bpw = sb // nw
    ch = 128
    nch = bpw // ch
    mesh = plsc.VectorSubcoreMesh(core_axis_name="c", subcore_axis_name="s")

    nbuf = 4
    lag = 2
    ngrp = nch // nbuf

    @functools.partial(
        pl.kernel,
        mesh=mesh,
        out_type=jax.ShapeDtypeStruct((sb, d), jnp.int32),
        scratch_types=[
            pltpu.VMEM((nbuf, ch), jnp.int32),
            pltpu.VMEM((nbuf, ch, d), jnp.int32),
            pltpu.SemaphoreType.DMA((nbuf,)),
            pltpu.SemaphoreType.DMA((nbuf,)),
            pltpu.SemaphoreType.DMA((nbuf,)),
        ],
    )
    def gather_k(idx_hbm, table_hbm, out_hbm, idx_v, rows_v, sem_i, sem_g, sem_w):
        wid = lax.axis_index("s") * nc + lax.axis_index("c")
        base = wid * bpw

        def idx_slice(c):
            return idx_hbm.at[pl.ds(base + c * ch, ch)]

        def out_slice(c):
            return out_hbm.at[pl.ds(base + c * ch, ch)]

        for b in range(nbuf):
            pltpu.async_copy(idx_slice(b), idx_v.at[b], sem_i.at[b])

        def group(g, carry):
            for b in range(nbuf):
                i = g * nbuf + b
                bl = (b - lag) % nbuf
                @pl.when(i >= nbuf)
                def _():
                    pltpu.make_async_copy(
                        rows_v.at[b], out_slice(i - nbuf), sem_w.at[b]
                    ).wait()

                pltpu.make_async_copy(idx_slice(i), idx_v.at[b], sem_i.at[b]).wait()
                pltpu.async_copy(
                    table_hbm.at[idx_v.at[b]], rows_v.at[b], sem_g.at[b]
                )

                @pl.when(i >= lag)
                def _():
                    pltpu.make_async_copy(
                        table_hbm.at[idx_v.at[bl]], rows_v.at[bl], sem_g.at[bl]
                    ).wait()
                    pltpu.async_copy(rows_v.at[bl], out_slice(i - lag), sem_w.at[bl])

                @pl.when((i >= lag) & (i - lag + nbuf < nch))
                def _():
                    pltpu.async_copy(
                        idx_slice(i - lag + nbuf), idx_v.at[bl], sem_i.at[bl]
                    )

            return carry

        lax.fori_loop(0, ngrp, group, 0)

        for k in range(lag):
            c = nch - lag + k
            bc = c % nbuf
            pltpu.make_async_copy(
                table_hbm.at[idx_v.at[bc]], rows_v.at[bc], sem_g.at[bc]
            ).wait()
            pltpu.async_copy(rows_v.at[bc], out_slice(c), sem_w.at[bc])
        for b in range(nbuf):
            pltpu.make_async_copy(
                rows_v.at[b], out_slice(nch - nbuf + b), sem_w.at[b]
            ).wait()

    return gather_k


def _tc_body(x_ref, g_ref, w_ref, b_ref, o_ref):
    dense = (
        jnp.dot(x_ref[...], w_ref[...], preferred_element_type=jnp.float32)
        + b_ref[...]
    )
    gi = g_ref[...]
    hi = jax.lax.bitcast_convert_type(
        jnp.bitwise_and(gi, jnp.int32(-65536)), jnp.float32
    )
    lo = jax.lax.bitcast_convert_type(jnp.left_shift(gi, 16), jnp.float32)
    o_ref[...] = dense + jnp.concatenate([hi, lo], axis=1)


def kernel(x, emb_table, W_epoch, W_cfg, b_cfg):
    x_flat = x.reshape(SB, IN_DIM)
    ids = x_flat[:, 0].astype(jnp.int32)
    t16 = emb_table.astype(jnp.bfloat16)
    au = jax.lax.bitcast_convert_type(t16[:, : OUT_DIM // 2], jnp.uint16)
    bu = jax.lax.bitcast_convert_type(t16[:, OUT_DIM // 2 :], jnp.uint16)
    table_pk = jax.lax.bitcast_convert_type(
        (au.astype(jnp.uint32) << 16) | bu.astype(jnp.uint32), jnp.int32
    )

    inv_std = 1.0 / math.sqrt(1.0 / 12.0)
    w_epoch_row = (W_epoch[:, 0] * inv_std)[None, :]
    b_all = (b_cfg - 0.5 * inv_std * W_epoch[:, 0])[None, :]
    w34 = jnp.concatenate(
        [jnp.zeros((1, OUT_DIM), jnp.float32), w_epoch_row, W_cfg.T], axis=0
    )

    gathered = _make_sc_gather(SB, OUT_DIM // 2)(ids, table_pk)

    bt = 1024
    out = pl.pallas_call(
        _tc_body,
        grid=(SB // bt,),
        in_specs=[
            pl.BlockSpec((bt, IN_DIM), lambda i: (i, 0)),
            pl.BlockSpec((bt, OUT_DIM // 2), lambda i: (i, 0)),
            pl.BlockSpec((IN_DIM, OUT_DIM), lambda i: (0, 0)),
            pl.BlockSpec((1, OUT_DIM), lambda i: (0, 0)),
        ],
        out_specs=pl.BlockSpec((bt, OUT_DIM), lambda i: (i, 0)),
        out_shape=jax.ShapeDtypeStruct((SB, OUT_DIM), jnp.float32),
    )(x_flat, gathered, w34, b_all)

    return out.reshape(SEQ, BATCH, OUT_DIM)


# speedup vs baseline: 2.6771x; 1.2361x over previous
import functools
import math

import jax
import jax.numpy as jnp
from jax import lax
from jax.experimental import pallas as pl
from jax.experimental.pallas import tpu as pltpu
from jax.experimental.pallas import tpu_sc as plsc

IN_DIM = 34
OUT_DIM = 256
SEQ = 2048
BATCH = 128
N_EMB = 1001
SB = SEQ * BATCH


def _make_sc_gather(sb, d):
    info = plsc.get_sparse_core_info()
    nc, ns = info.num_cores, info.num_subcores
    nw = nc * ns
    bpw = sb // nw
    ch = 128
    nch = bpw // ch
    mesh = plsc.VectorSubcoreMesh(core_axis_name="c", subcore_axis_name="s")

    nbuf = 4
    lag = 2
    ngrp = nch // nbuf

    @functools.partial(
        pl.kernel,
        mesh=mesh,
        out_type=jax.ShapeDtypeStruct((sb, d), jnp.int32),
        scratch_types=[
            pltpu.VMEM((nbuf, ch), jnp.int32),
            pltpu.VMEM((nbuf, ch, d), jnp.int32),
            pltpu.VMEM_SHARED((N_EMB, d), jnp.int32),
            pltpu.SemaphoreType.DMA((nbuf,)),
            pltpu.SemaphoreType.DMA((nbuf,)),
            pltpu.SemaphoreType.DMA((nbuf,)),
        ],
    )
    def gather_k(
        idx_hbm, table_hbm, out_hbm, idx_v, rows_v, table_sh, sem_i, sem_g, sem_w
    ):
        wid = lax.axis_index("s") * nc + lax.axis_index("c")
        base = wid * bpw

        @pl.when(lax.axis_index("s") == 0)
        def _():
            pltpu.sync_copy(table_hbm, table_sh)

        plsc.subcore_barrier()

        def idx_slice(c):
            return idx_hbm.at[pl.ds(base + c * ch, ch)]

        def out_slice(c):
            return out_hbm.at[pl.ds(base + c * ch, ch)]

        for b in range(nbuf):
            pltpu.async_copy(idx_slice(b), idx_v.at[b], sem_i.at[b])

        def group(g, carry):
            for b in range(nbuf):
                i = g * nbuf + b
                bl = (b - lag) % nbuf
                @pl.when(i >= nbuf)
                def _():
                    pltpu.make_async_copy(
                        rows_v.at[b], out_slice(i - nbuf), sem_w.at[b]
                    ).wait()

                pltpu.make_async_copy(idx_slice(i), idx_v.at[b], sem_i.at[b]).wait()
                pltpu.async_copy(
                    table_sh.at[idx_v.at[b]], rows_v.at[b], sem_g.at[b]
                )

                @pl.when(i >= lag)
                def _():
                    pltpu.make_async_copy(
                        table_sh.at[idx_v.at[bl]], rows_v.at[bl], sem_g.at[bl]
                    ).wait()
                    pltpu.async_copy(rows_v.at[bl], out_slice(i - lag), sem_w.at[bl])

                @pl.when((i >= lag) & (i - lag + nbuf < nch))
                def _():
                    pltpu.async_copy(
                        idx_slice(i - lag + nbuf), idx_v.at[bl], sem_i.at[bl]
                    )

            return carry

        lax.fori_loop(0, ngrp, group, 0)

        for k in range(lag):
            c = nch - lag + k
            bc = c % nbuf
            pltpu.make_async_copy(
                table_sh.at[idx_v.at[bc]], rows_v.at[bc], sem_g.at[bc]
            ).wait()
            pltpu.async_copy(rows_v.at[bc], out_slice(c), sem_w.at[bc])
        for b in range(nbuf):
            pltpu.make_async_copy(
                rows_v.at[b], out_slice(nch - nbuf + b), sem_w.at[b]
            ).wait()

    return gather_k


def _tc_body(x_ref, g_ref, w_ref, b_ref, o_ref):
    dense = (
        jnp.dot(x_ref[...], w_ref[...], preferred_element_type=jnp.float32)
        + b_ref[...]
    )
    gi = g_ref[...]
    hi = jax.lax.bitcast_convert_type(
        jnp.bitwise_and(gi, jnp.int32(-65536)), jnp.float32
    )
    lo = jax.lax.bitcast_convert_type(jnp.left_shift(gi, 16), jnp.float32)
    o_ref[...] = dense + jnp.concatenate([hi, lo], axis=1)


def kernel(x, emb_table, W_epoch, W_cfg, b_cfg):
    x_flat = x.reshape(SB, IN_DIM)
    ids = x_flat[:, 0].astype(jnp.int32)
    t16 = emb_table.astype(jnp.bfloat16)
    au = jax.lax.bitcast_convert_type(t16[:, : OUT_DIM // 2], jnp.uint16)
    bu = jax.lax.bitcast_convert_type(t16[:, OUT_DIM // 2 :], jnp.uint16)
    table_pk = jax.lax.bitcast_convert_type(
        (au.astype(jnp.uint32) << 16) | bu.astype(jnp.uint32), jnp.int32
    )

    inv_std = 1.0 / math.sqrt(1.0 / 12.0)
    w_epoch_row = (W_epoch[:, 0] * inv_std)[None, :]
    b_all = (b_cfg - 0.5 * inv_std * W_epoch[:, 0])[None, :]
    w34 = jnp.concatenate(
        [jnp.zeros((1, OUT_DIM), jnp.float32), w_epoch_row, W_cfg.T], axis=0
    )

    gathered = _make_sc_gather(SB, OUT_DIM // 2)(ids, table_pk)

    bt = 1024
    out = pl.pallas_call(
        _tc_body,
        grid=(SB // bt,),
        in_specs=[
            pl.BlockSpec((bt, IN_DIM), lambda i: (i, 0)),
            pl.BlockSpec((bt, OUT_DIM // 2), lambda i: (i, 0)),
            pl.BlockSpec((IN_DIM, OUT_DIM), lambda i: (0, 0)),
            pl.BlockSpec((1, OUT_DIM), lambda i: (0, 0)),
        ],
        out_specs=pl.BlockSpec((bt, OUT_DIM), lambda i: (i, 0)),
        out_shape=jax.ShapeDtypeStruct((SB, OUT_DIM), jnp.float32),
    )(x_flat, gathered, w34, b_all)

    return out.reshape(SEQ, BATCH, OUT_DIM)


# speedup vs baseline: 3.2591x; 1.2174x over previous
import functools
import math

import jax
import jax.numpy as jnp
from jax import lax
from jax.experimental import pallas as pl
from jax.experimental.pallas import tpu as pltpu
from jax.experimental.pallas import tpu_sc as plsc

IN_DIM = 34
OUT_DIM = 256
SEQ = 2048
BATCH = 128
N_EMB = 1001
SB = SEQ * BATCH


def _make_sc_gather(sb, d):
    info = plsc.get_sparse_core_info()
    nc, ns = info.num_cores, info.num_subcores
    nw = nc * ns
    bpw = sb // nw
    ch = 128
    nch = bpw // ch
    mesh = plsc.VectorSubcoreMesh(core_axis_name="c", subcore_axis_name="s")

    nbuf = 4
    lag = 2
    ngrp = nch // nbuf

    @functools.partial(
        pl.kernel,
        mesh=mesh,
        out_type=jax.ShapeDtypeStruct((sb, d), jnp.int32),
        scratch_types=[
            pltpu.VMEM((nbuf, ch), jnp.int32),
            pltpu.VMEM((nbuf, ch, d), jnp.int32),
            pltpu.VMEM_SHARED((N_EMB, d), jnp.int32),
            pltpu.SemaphoreType.DMA((nbuf,)),
            pltpu.SemaphoreType.DMA((nbuf,)),
            pltpu.SemaphoreType.DMA((nbuf,)),
        ],
    )
    def gather_k(
        idx_hbm, table_hbm, out_hbm, idx_v, rows_v, table_sh, sem_i, sem_g, sem_w
    ):
        wid = lax.axis_index("s") * nc + lax.axis_index("c")
        base = wid * bpw

        @pl.when(lax.axis_index("s") == 0)
        def _():
            pltpu.sync_copy(table_hbm, table_sh)

        plsc.subcore_barrier()

        def idx_slice(c):
            return idx_hbm.at[pl.ds(base + c * ch, ch)]

        def out_slice(c):
            return out_hbm.at[pl.ds(base + c * ch, ch)]

        for b in range(nbuf):
            pltpu.async_copy(idx_slice(b), idx_v.at[b], sem_i.at[b])

        def group(g, carry):
            for b in range(nbuf):
                i = g * nbuf + b
                bl = (b - lag) % nbuf
                @pl.when(i >= nbuf)
                def _():
                    pltpu.make_async_copy(
                        rows_v.at[b], out_slice(i - nbuf), sem_w.at[b]
                    ).wait()

                pltpu.make_async_copy(idx_slice(i), idx_v.at[b], sem_i.at[b]).wait()
                pltpu.async_copy(
                    table_sh.at[idx_v.at[b]], rows_v.at[b], sem_g.at[b]
                )

                @pl.when(i >= lag)
                def _():
                    pltpu.make_async_copy(
                        table_sh.at[idx_v.at[bl]], rows_v.at[bl], sem_g.at[bl]
                    ).wait()
                    pltpu.async_copy(rows_v.at[bl], out_slice(i - lag), sem_w.at[bl])

                @pl.when((i >= lag) & (i - lag + nbuf < nch))
                def _():
                    pltpu.async_copy(
                        idx_slice(i - lag + nbuf), idx_v.at[bl], sem_i.at[bl]
                    )

            return carry

        lax.fori_loop(0, ngrp, group, 0)

        for k in range(lag):
            c = nch - lag + k
            bc = c % nbuf
            pltpu.make_async_copy(
                table_sh.at[idx_v.at[bc]], rows_v.at[bc], sem_g.at[bc]
            ).wait()
            pltpu.async_copy(rows_v.at[bc], out_slice(c), sem_w.at[bc])
        for b in range(nbuf):
            pltpu.make_async_copy(
                rows_v.at[b], out_slice(nch - nbuf + b), sem_w.at[b]
            ).wait()

    return gather_k


def _tc_body(x_ref, g_ref, w_ref, b_ref, o_ref):
    dense = (
        jnp.dot(x_ref[...], w_ref[...], preferred_element_type=jnp.float32)
        + b_ref[...]
    )
    gi = g_ref[...]
    hi = jax.lax.bitcast_convert_type(
        jnp.bitwise_and(gi, jnp.int32(-65536)), jnp.float32
    )
    lo = jax.lax.bitcast_convert_type(jnp.left_shift(gi, 16), jnp.float32)
    o_ref[...] = dense + jnp.concatenate([hi, lo], axis=1)


def kernel(x, emb_table, W_epoch, W_cfg, b_cfg):
    x_flat = x.reshape(SB, IN_DIM)
    ids = x_flat[:, 0].astype(jnp.int32)
    t16 = emb_table.astype(jnp.bfloat16)
    au = jax.lax.bitcast_convert_type(t16[:, : OUT_DIM // 2], jnp.uint16)
    bu = jax.lax.bitcast_convert_type(t16[:, OUT_DIM // 2 :], jnp.uint16)
    table_pk = jax.lax.bitcast_convert_type(
        (au.astype(jnp.uint32) << 16) | bu.astype(jnp.uint32), jnp.int32
    )

    inv_std = 1.0 / math.sqrt(1.0 / 12.0)
    w_epoch_row = (W_epoch[:, 0] * inv_std)[None, :]
    b_all = (b_cfg - 0.5 * inv_std * W_epoch[:, 0])[None, :]
    w34 = jnp.concatenate(
        [jnp.zeros((1, OUT_DIM), jnp.float32), w_epoch_row, W_cfg.T], axis=0
    )

    gathered = _make_sc_gather(SB, OUT_DIM // 2)(ids, table_pk)

    bt = 2048
    out = pl.pallas_call(
        _tc_body,
        grid=(SB // bt,),
        in_specs=[
            pl.BlockSpec((bt, IN_DIM), lambda i: (i, 0)),
            pl.BlockSpec((bt, OUT_DIM // 2), lambda i: (i, 0)),
            pl.BlockSpec((IN_DIM, OUT_DIM), lambda i: (0, 0)),
            pl.BlockSpec((1, OUT_DIM), lambda i: (0, 0)),
        ],
        out_specs=pl.BlockSpec((bt, OUT_DIM), lambda i: (i, 0)),
        out_shape=jax.ShapeDtypeStruct((SB, OUT_DIM), jnp.float32),
    )(x_flat, gathered, w34, b_all)

    return out.reshape(SEQ, BATCH, OUT_DIM)


# speedup vs baseline: 3.4796x; 1.0676x over previous
import functools
import math

import jax
import jax.numpy as jnp
from jax import lax
from jax.experimental import pallas as pl
from jax.experimental.pallas import tpu as pltpu
from jax.experimental.pallas import tpu_sc as plsc

IN_DIM = 34
OUT_DIM = 256
SEQ = 2048
BATCH = 128
N_EMB = 1001
SB = SEQ * BATCH


def _make_sc_gather(sb, d):
    info = plsc.get_sparse_core_info()
    nc, ns = info.num_cores, info.num_subcores
    nw = nc * ns
    bpw = sb // nw
    ch = 128
    nch = bpw // ch
    mesh = plsc.VectorSubcoreMesh(core_axis_name="c", subcore_axis_name="s")

    nbuf = 4
    lag = 2
    ngrp = nch // nbuf

    @functools.partial(
        pl.kernel,
        mesh=mesh,
        out_type=jax.ShapeDtypeStruct((sb, d), jnp.int32),
        scratch_types=[
            pltpu.VMEM((nbuf, ch), jnp.int32),
            pltpu.VMEM((nbuf, ch, d), jnp.int32),
            pltpu.VMEM_SHARED((N_EMB, d), jnp.int32),
            pltpu.SemaphoreType.DMA((nbuf,)),
            pltpu.SemaphoreType.DMA((nbuf,)),
            pltpu.SemaphoreType.DMA((nbuf,)),
        ],
    )
    def gather_k(
        idx_hbm, table_hbm, out_hbm, idx_v, rows_v, table_sh, sem_i, sem_g, sem_w
    ):
        wid = lax.axis_index("s") * nc + lax.axis_index("c")
        base = wid * bpw

        @pl.when(lax.axis_index("s") == 0)
        def _():
            pltpu.sync_copy(table_hbm, table_sh)

        plsc.subcore_barrier()

        def idx_slice(c):
            return idx_hbm.at[pl.ds(base + c * ch, ch)]

        def out_slice(c):
            return out_hbm.at[pl.ds(base + c * ch, ch)]

        for b in range(nbuf):
            pltpu.async_copy(idx_slice(b), idx_v.at[b], sem_i.at[b])

        def group(g, carry):
            for b in range(nbuf):
                i = g * nbuf + b
                bl = (b - lag) % nbuf
                @pl.when(i >= nbuf)
                def _():
                    pltpu.make_async_copy(
                        rows_v.at[b], out_slice(i - nbuf), sem_w.at[b]
                    ).wait()

                pltpu.make_async_copy(idx_slice(i), idx_v.at[b], sem_i.at[b]).wait()
                pltpu.async_copy(
                    table_sh.at[idx_v.at[b]], rows_v.at[b], sem_g.at[b]
                )

                @pl.when(i >= lag)
                def _():
                    pltpu.make_async_copy(
                        table_sh.at[idx_v.at[bl]], rows_v.at[bl], sem_g.at[bl]
                    ).wait()
                    pltpu.async_copy(rows_v.at[bl], out_slice(i - lag), sem_w.at[bl])

                @pl.when((i >= lag) & (i - lag + nbuf < nch))
                def _():
                    pltpu.async_copy(
                        idx_slice(i - lag + nbuf), idx_v.at[bl], sem_i.at[bl]
                    )

            return carry

        lax.fori_loop(0, ngrp, group, 0)

        for k in range(lag):
            c = nch - lag + k
            bc = c % nbuf
            pltpu.make_async_copy(
                table_sh.at[idx_v.at[bc]], rows_v.at[bc], sem_g.at[bc]
            ).wait()
            pltpu.async_copy(rows_v.at[bc], out_slice(c), sem_w.at[bc])
        for b in range(nbuf):
            pltpu.make_async_copy(
                rows_v.at[b], out_slice(nch - nbuf + b), sem_w.at[b]
            ).wait()

    return gather_k


def _tc_body(x_ref, g_ref, w_ref, b_ref, o_ref):
    dense = (
        jnp.dot(x_ref[...], w_ref[...], preferred_element_type=jnp.float32)
        + b_ref[...]
    )
    gi = g_ref[...]
    hi = jax.lax.bitcast_convert_type(
        jnp.bitwise_and(gi, jnp.int32(-65536)), jnp.float32
    )
    lo = jax.lax.bitcast_convert_type(jnp.left_shift(gi, 16), jnp.float32)
    o_ref[...] = dense + jnp.concatenate([hi, lo], axis=1)


def kernel(x, emb_table, W_epoch, W_cfg, b_cfg):
    x_flat = x.reshape(SB, IN_DIM)
    ids = x_flat[:, 0].astype(jnp.int32)
    t16 = emb_table.astype(jnp.bfloat16)
    au = jax.lax.bitcast_convert_type(t16[:, : OUT_DIM // 2], jnp.uint16)
    bu = jax.lax.bitcast_convert_type(t16[:, OUT_DIM // 2 :], jnp.uint16)
    table_pk = jax.lax.bitcast_convert_type(
        (au.astype(jnp.uint32) << 16) | bu.astype(jnp.uint32), jnp.int32
    )

    inv_std = 1.0 / math.sqrt(1.0 / 12.0)
    w_epoch_row = (W_epoch[:, 0] * inv_std)[None, :]
    b_all = (b_cfg - 0.5 * inv_std * W_epoch[:, 0])[None, :]
    w34 = jnp.concatenate(
        [jnp.zeros((1, OUT_DIM), jnp.float32), w_epoch_row, W_cfg.T], axis=0
    )

    gathered = _make_sc_gather(SB, OUT_DIM // 2)(ids, table_pk)

    bt = 4096
    out = pl.pallas_call(
        _tc_body,
        grid=(SB // bt,),
        in_specs=[
            pl.BlockSpec((bt, IN_DIM), lambda i: (i, 0)),
            pl.BlockSpec((bt, OUT_DIM // 2), lambda i: (i, 0)),
            pl.BlockSpec((IN_DIM, OUT_DIM), lambda i: (0, 0)),
            pl.BlockSpec((1, OUT_DIM), lambda i: (0, 0)),
        ],
        out_specs=pl.BlockSpec((bt, OUT_DIM), lambda i: (i, 0)),
        out_shape=jax.ShapeDtypeStruct((SB, OUT_DIM), jnp.float32),
    )(x_flat, gathered, w34, b_all)

    return out.reshape(SEQ, BATCH, OUT_DIM)


# speedup vs baseline: 3.5243x; 1.0129x over previous
import functools
import math

import jax
import jax.numpy as jnp
from jax import lax
from jax.experimental import pallas as pl
from jax.experimental.pallas import tpu as pltpu
from jax.experimental.pallas import tpu_sc as plsc

IN_DIM = 34
OUT_DIM = 256
SEQ = 2048
BATCH = 128
N_EMB = 1001
SB = SEQ * BATCH


def _make_sc_gather(sb, d):
    info = plsc.get_sparse_core_info()
    nc, ns = info.num_cores, info.num_subcores
    nw = nc * ns
    bpw = sb // nw
    ch = 128
    nch = bpw // ch
    mesh = plsc.VectorSubcoreMesh(core_axis_name="c", subcore_axis_name="s")

    nbuf = 4
    lag = 2
    ngrp = nch // nbuf

    @functools.partial(
        pl.kernel,
        mesh=mesh,
        out_type=jax.ShapeDtypeStruct((sb, d), jnp.int32),
        scratch_types=[
            pltpu.VMEM((nbuf, ch), jnp.int32),
            pltpu.VMEM((nbuf, ch, d), jnp.int32),
            pltpu.VMEM_SHARED((N_EMB, d), jnp.int32),
            pltpu.SemaphoreType.DMA((nbuf,)),
            pltpu.SemaphoreType.DMA((nbuf,)),
            pltpu.SemaphoreType.DMA((nbuf,)),
        ],
    )
    def gather_k(
        idx_hbm, table_hbm, out_hbm, idx_v, rows_v, table_sh, sem_i, sem_g, sem_w
    ):
        wid = lax.axis_index("s") * nc + lax.axis_index("c")
        base = wid * bpw

        @pl.when(lax.axis_index("s") == 0)
        def _():
            pltpu.sync_copy(table_hbm, table_sh)

        plsc.subcore_barrier()

        def idx_slice(c):
            return idx_hbm.at[pl.ds(base + c * ch, ch)]

        def out_slice(c):
            return out_hbm.at[pl.ds(base + c * ch, ch)]

        for b in range(nbuf):
            pltpu.async_copy(idx_slice(b), idx_v.at[b], sem_i.at[b])

        def group(g, carry):
            for b in range(nbuf):
                i = g * nbuf + b
                bl = (b - lag) % nbuf
                @pl.when(i >= nbuf)
                def _():
                    pltpu.make_async_copy(
                        rows_v.at[b], out_slice(i - nbuf), sem_w.at[b]
                    ).wait()

                pltpu.make_async_copy(idx_slice(i), idx_v.at[b], sem_i.at[b]).wait()
                pltpu.async_copy(
                    table_sh.at[idx_v.at[b]], rows_v.at[b], sem_g.at[b]
                )

                @pl.when(i >= lag)
                def _():
                    pltpu.make_async_copy(
                        table_sh.at[idx_v.at[bl]], rows_v.at[bl], sem_g.at[bl]
                    ).wait()
                    pltpu.async_copy(rows_v.at[bl], out_slice(i - lag), sem_w.at[bl])

                @pl.when((i >= lag) & (i - lag + nbuf < nch))
                def _():
                    pltpu.async_copy(
                        idx_slice(i - lag + nbuf), idx_v.at[bl], sem_i.at[bl]
                    )

            return carry

        lax.fori_loop(0, ngrp, group, 0)

        for k in range(lag):
            c = nch - lag + k
            bc = c % nbuf
            pltpu.make_async_copy(
                table_sh.at[idx_v.at[bc]], rows_v.at[bc], sem_g.at[bc]
            ).wait()
            pltpu.async_copy(rows_v.at[bc], out_slice(c), sem_w.at[bc])
        for b in range(nbuf):
            pltpu.make_async_copy(
                rows_v.at[b], out_slice(nch - nbuf + b), sem_w.at[b]
            ).wait()

    return gather_k


def _tc_body(x_ref, g_ref, w_ref, b_ref, o_ref):
    dense = (
        jnp.dot(x_ref[...], w_ref[...], preferred_element_type=jnp.float32)
        + b_ref[...]
    )
    gi = g_ref[...]
    hi = jax.lax.bitcast_convert_type(
        jnp.bitwise_and(gi, jnp.int32(-65536)), jnp.float32
    )
    lo = jax.lax.bitcast_convert_type(jnp.left_shift(gi, 16), jnp.float32)
    o_ref[...] = dense + jnp.concatenate([hi, lo], axis=1)


def kernel(x, emb_table, W_epoch, W_cfg, b_cfg):
    x_flat = x.reshape(SB, IN_DIM)
    ids = x_flat[:, 0].astype(jnp.int32)
    t16 = emb_table.astype(jnp.bfloat16)
    au = jax.lax.bitcast_convert_type(t16[:, : OUT_DIM // 2], jnp.uint16)
    bu = jax.lax.bitcast_convert_type(t16[:, OUT_DIM // 2 :], jnp.uint16)
    table_pk = jax.lax.bitcast_convert_type(
        (au.astype(jnp.uint32) << 16) | bu.astype(jnp.uint32), jnp.int32
    )

    inv_std = 1.0 / math.sqrt(1.0 / 12.0)
    w_epoch_row = (W_epoch[:, 0] * inv_std)[None, :]
    b_all = (b_cfg - 0.5 * inv_std * W_epoch[:, 0])[None, :]
    w34 = jnp.concatenate(
        [jnp.zeros((1, OUT_DIM), jnp.float32), w_epoch_row, W_cfg.T], axis=0
    )

    gathered = _make_sc_gather(SB, OUT_DIM // 2)(ids, table_pk)

    bt = 8192
    out = pl.pallas_call(
        _tc_body,
        grid=(SB // bt,),
        in_specs=[
            pl.BlockSpec((bt, IN_DIM), lambda i: (i, 0)),
            pl.BlockSpec((bt, OUT_DIM // 2), lambda i: (i, 0)),
            pl.BlockSpec((IN_DIM, OUT_DIM), lambda i: (0, 0)),
            pl.BlockSpec((1, OUT_DIM), lambda i: (0, 0)),
        ],
        out_specs=pl.BlockSpec((bt, OUT_DIM), lambda i: (i, 0)),
        out_shape=jax.ShapeDtypeStruct((SB, OUT_DIM), jnp.float32),
    )(x_flat, gathered, w34, b_all)

    return out.reshape(SEQ, BATCH, OUT_DIM)
